# baseline (speedup 1.0000x reference)
import jax
import jax.numpy as jnp
from jax import lax
from jax.experimental import pallas as pl
from jax.experimental.pallas import tpu as pltpu
from jax.experimental.pallas import tpu_sc as plsc

_D = 512


def _body(l1_hbm, l0_hbm, out_hbm, buf, sem):
    cid = lax.axis_index("c")

    @pl.when(cid == 0)
    def _():
        pltpu.sync_copy(l1_hbm.at[pl.ds(0, 128)], buf)
        pltpu.sync_copy(buf, out_hbm.at[pl.ds(0, 128)])

    @pl.when(cid == 1)
    def _():
        pairs = [
            pltpu.make_async_copy(
                l0_hbm.at[pl.ds(4 * k, 2)], buf.at[pl.ds(2 * k, 2)], sem
            )
            for k in range(64)
        ]
        for c in pairs:
            c.start()
        for c in pairs:
            c.wait()
        pltpu.sync_copy(buf, out_hbm.at[pl.ds(128, 128)])


@jax.jit
def kernel(layer1, layer0):
    mesh = plsc.ScalarSubcoreMesh(axis_name="c", num_cores=2)
    f = pl.kernel(
        _body,
        out_type=jax.ShapeDtypeStruct((256, _D), jnp.float32),
        mesh=mesh,
        scratch_types=[
            pltpu.VMEM_SHARED((128, _D), jnp.float32),
            pltpu.SemaphoreType.DMA,
        ],
    )
    return f(layer1, layer0)


# speedup vs baseline: 1.0497x; 1.0497x over previous
import jax
import jax.numpy as jnp
from jax.experimental import pallas as pl
from jax.experimental.pallas import tpu as pltpu
from jax.experimental.pallas import tpu_sc as plsc

_D = 512
_GROUPS = 4
_PAIRS_PER_GROUP = 16


def _body(l1_hbm, l0_hbm, out_hbm, buf, s1a, s1b, sg, so):
    c1a = pltpu.make_async_copy(l1_hbm.at[pl.ds(0, 64)], buf.at[pl.ds(0, 64)], s1a)
    c1b = pltpu.make_async_copy(l1_hbm.at[pl.ds(64, 64)], buf.at[pl.ds(64, 64)], s1b)
    c1a.start()
    c1b.start()
    groups = []
    for g in range(_GROUPS):
        grp = []
        for j in range(_PAIRS_PER_GROUP):
            k = g * _PAIRS_PER_GROUP + j
            grp.append(
                pltpu.make_async_copy(
                    l0_hbm.at[pl.ds(4 * k, 2)],
                    buf.at[pl.ds(128 + 2 * k, 2)],
                    sg[g],
                )
            )
        groups.append(grp)
    for grp in groups:
        for c in grp:
            c.start()

    outs = []
    c1a.wait()
    o = pltpu.make_async_copy(buf.at[pl.ds(0, 64)], out_hbm.at[pl.ds(0, 64)], so)
    o.start()
    outs.append(o)
    c1b.wait()
    o = pltpu.make_async_copy(buf.at[pl.ds(64, 64)], out_hbm.at[pl.ds(64, 64)], so)
    o.start()
    outs.append(o)
    for g, grp in enumerate(groups):
        for c in grp:
            c.wait()
        base = 128 + g * 2 * _PAIRS_PER_GROUP
        o = pltpu.make_async_copy(
            buf.at[pl.ds(base, 2 * _PAIRS_PER_GROUP)],
            out_hbm.at[pl.ds(base, 2 * _PAIRS_PER_GROUP)],
            so,
        )
        o.start()
        outs.append(o)
    for o in outs:
        o.wait()


@jax.jit
def kernel(layer1, layer0):
    mesh = plsc.ScalarSubcoreMesh(axis_name="c", num_cores=1)
    f = pl.kernel(
        _body,
        out_type=jax.ShapeDtypeStruct((256, _D), jnp.float32),
        mesh=mesh,
        scratch_types=[
            pltpu.VMEM_SHARED((256, _D), jnp.float32),
            pltpu.SemaphoreType.DMA,
            pltpu.SemaphoreType.DMA,
            [pltpu.SemaphoreType.DMA] * _GROUPS,
            pltpu.SemaphoreType.DMA,
        ],
    )
    return f(layer1, layer0)
